# initial kernel scaffold (unmeasured)
import jax
import jax.numpy as jnp
from jax import lax
from jax.experimental import pallas as pl
from jax.experimental.pallas import tpu as pltpu


def kernel(
    x,
):
    def body(*refs):
        pass

    out_shape = jax.ShapeDtypeStruct(..., jnp.float32)
    return pl.pallas_call(body, out_shape=out_shape)(...)


# baseline (device time: 2920815 ns/iter reference)
import jax
import jax.numpy as jnp
from jax import lax
from jax.experimental import pallas as pl
from jax.experimental.pallas import tpu as pltpu

N_DEV = 16

_PERM = [0, 4, 8, 12, 13, 9, 5, 1, 2, 6, 10, 14, 15, 11, 7, 3]
_INV = [0] * N_DEV
for _r, _l in enumerate(_PERM):
    _INV[_l] = _r


def kernel(x):
    m_per, n = x.shape

    perm_t = jnp.array(_PERM, dtype=jnp.int32)
    inv_t = jnp.array(_INV, dtype=jnp.int32)

    my = lax.axis_index("i")
    r = inv_t[my]
    origins_fw = perm_t[(r - jnp.arange(N_DEV)) % N_DEV]
    right = perm_t[(r + 1) % N_DEV]
    left = perm_t[(r - 1) % N_DEV]
    meta = jnp.concatenate(
        [origins_fw, right[None], left[None]]
    ).astype(jnp.int32)

    def body(meta_ref, x_ref, out_ref, send_sems, recv_sems, copy_sem):
        right = meta_ref[N_DEV]
        left = meta_ref[N_DEV + 1]

        barrier_sem = pltpu.get_barrier_semaphore()
        for nbr in (left, right):
            pl.semaphore_signal(
                barrier_sem,
                inc=1,
                device_id=(nbr,),
                device_id_type=pl.DeviceIdType.MESH,
            )
        pl.semaphore_wait(barrier_sem, 2)

        own = meta_ref[0]
        cp = pltpu.make_async_copy(
            x_ref, out_ref.at[pl.ds(own * m_per, m_per)], copy_sem
        )
        cp.start()
        cp.wait()

        for h in range(N_DEV - 1):
            org = meta_ref[h]
            rdma = pltpu.make_async_remote_copy(
                src_ref=out_ref.at[pl.ds(org * m_per, m_per)],
                dst_ref=out_ref.at[pl.ds(org * m_per, m_per)],
                send_sem=send_sems.at[h],
                recv_sem=recv_sems.at[h],
                device_id=(right,),
                device_id_type=pl.DeviceIdType.MESH,
            )
            rdma.start()
            rdma.wait()

    return pl.pallas_call(
        body,
        out_shape=jax.ShapeDtypeStruct((N_DEV * m_per, n), x.dtype),
        in_specs=[
            pl.BlockSpec(memory_space=pltpu.MemorySpace.SMEM),
            pl.BlockSpec(memory_space=pltpu.MemorySpace.HBM),
        ],
        out_specs=pl.BlockSpec(memory_space=pltpu.MemorySpace.HBM),
        scratch_shapes=[
            pltpu.SemaphoreType.DMA((N_DEV - 1,)),
            pltpu.SemaphoreType.DMA((N_DEV - 1,)),
            pltpu.SemaphoreType.DMA,
        ],
        compiler_params=pltpu.CompilerParams(collective_id=0),
    )(meta, x)


# device time: 1571305 ns/iter; 1.8588x vs baseline; 1.8588x over previous
import jax
import jax.numpy as jnp
from jax import lax
from jax.experimental import pallas as pl
from jax.experimental.pallas import tpu as pltpu

N_DEV = 16

_PERM = [0, 4, 8, 12, 13, 9, 5, 1, 2, 6, 10, 14, 15, 11, 7, 3]
_INV = [0] * N_DEV
for _r, _l in enumerate(_PERM):
    _INV[_l] = _r


def kernel(x):
    m_per, n = x.shape
    half = m_per // 2

    perm_t = jnp.array(_PERM, dtype=jnp.int32)
    inv_t = jnp.array(_INV, dtype=jnp.int32)

    my = lax.axis_index("i")
    r = inv_t[my]
    origins_fw = perm_t[(r - jnp.arange(N_DEV)) % N_DEV]
    origins_bw = perm_t[(r + jnp.arange(N_DEV)) % N_DEV]
    right = perm_t[(r + 1) % N_DEV]
    left = perm_t[(r - 1) % N_DEV]
    meta = jnp.concatenate(
        [origins_fw, origins_bw, right[None], left[None]]
    ).astype(jnp.int32)

    def body(
        meta_ref,
        x_ref,
        out_ref,
        send_f,
        recv_f,
        send_b,
        recv_b,
        copy_sem,
    ):
        right = meta_ref[2 * N_DEV]
        left = meta_ref[2 * N_DEV + 1]

        barrier_sem = pltpu.get_barrier_semaphore()
        for nbr in (left, right):
            pl.semaphore_signal(
                barrier_sem,
                inc=1,
                device_id=(nbr,),
                device_id_type=pl.DeviceIdType.MESH,
            )
        pl.semaphore_wait(barrier_sem, 2)

        own = meta_ref[0]
        cp = pltpu.make_async_copy(
            x_ref, out_ref.at[pl.ds(own * m_per, m_per)], copy_sem
        )
        cp.start()
        cp.wait()

        for h in range(N_DEV - 1):
            of = meta_ref[h]
            ob = meta_ref[N_DEV + h]
            rdma_f = pltpu.make_async_remote_copy(
                src_ref=out_ref.at[pl.ds(of * m_per, half)],
                dst_ref=out_ref.at[pl.ds(of * m_per, half)],
                send_sem=send_f.at[h],
                recv_sem=recv_f.at[h],
                device_id=(right,),
                device_id_type=pl.DeviceIdType.MESH,
            )
            rdma_b = pltpu.make_async_remote_copy(
                src_ref=out_ref.at[pl.ds(ob * m_per + half, half)],
                dst_ref=out_ref.at[pl.ds(ob * m_per + half, half)],
                send_sem=send_b.at[h],
                recv_sem=recv_b.at[h],
                device_id=(left,),
                device_id_type=pl.DeviceIdType.MESH,
            )
            rdma_f.start()
            rdma_b.start()
            rdma_f.wait()
            rdma_b.wait()

    return pl.pallas_call(
        body,
        out_shape=jax.ShapeDtypeStruct((N_DEV * m_per, n), x.dtype),
        in_specs=[
            pl.BlockSpec(memory_space=pltpu.MemorySpace.SMEM),
            pl.BlockSpec(memory_space=pltpu.MemorySpace.HBM),
        ],
        out_specs=pl.BlockSpec(memory_space=pltpu.MemorySpace.HBM),
        scratch_shapes=[
            pltpu.SemaphoreType.DMA((N_DEV - 1,)),
            pltpu.SemaphoreType.DMA((N_DEV - 1,)),
            pltpu.SemaphoreType.DMA((N_DEV - 1,)),
            pltpu.SemaphoreType.DMA((N_DEV - 1,)),
            pltpu.SemaphoreType.DMA,
        ],
        compiler_params=pltpu.CompilerParams(collective_id=0),
    )(meta, x)


# device time: 1569101 ns/iter; 1.8615x vs baseline; 1.0014x over previous
import jax
import jax.numpy as jnp
from jax import lax
from jax.experimental import pallas as pl
from jax.experimental.pallas import tpu as pltpu

N_DEV = 16

_PERM = [0, 4, 8, 12, 13, 9, 5, 1, 2, 6, 10, 14, 15, 11, 7, 3]
_INV = [0] * N_DEV
for _r, _l in enumerate(_PERM):
    _INV[_l] = _r


def kernel(x):
    m_per, n = x.shape
    half = m_per // 2

    perm_t = jnp.array(_PERM, dtype=jnp.int32)
    inv_t = jnp.array(_INV, dtype=jnp.int32)

    my = lax.axis_index("i")
    r = inv_t[my]
    origins_fw = perm_t[(r - jnp.arange(N_DEV)) % N_DEV]
    origins_bw = perm_t[(r + jnp.arange(N_DEV)) % N_DEV]
    right = perm_t[(r + 1) % N_DEV]
    left = perm_t[(r - 1) % N_DEV]
    meta = jnp.concatenate(
        [origins_fw, origins_bw, right[None], left[None]]
    ).astype(jnp.int32)

    def body(
        meta_ref,
        x_ref,
        out_ref,
        send_f,
        recv_f,
        send_b,
        recv_b,
        copy_sem,
    ):
        right = meta_ref[2 * N_DEV]
        left = meta_ref[2 * N_DEV + 1]

        barrier_sem = pltpu.get_barrier_semaphore()
        for nbr in (left, right):
            pl.semaphore_signal(
                barrier_sem,
                inc=1,
                device_id=(nbr,),
                device_id_type=pl.DeviceIdType.MESH,
            )
        pl.semaphore_wait(barrier_sem, 2)

        own = meta_ref[0]
        cp = pltpu.make_async_copy(
            x_ref, out_ref.at[pl.ds(own * m_per, m_per)], copy_sem
        )
        cp.start()

        rdmas = []
        for h in range(N_DEV - 1):
            of = meta_ref[h]
            ob = meta_ref[N_DEV + h]
            src_f = (
                x_ref.at[pl.ds(0, half)]
                if h == 0
                else out_ref.at[pl.ds(of * m_per, half)]
            )
            src_b = (
                x_ref.at[pl.ds(half, half)]
                if h == 0
                else out_ref.at[pl.ds(ob * m_per + half, half)]
            )
            rdma_f = pltpu.make_async_remote_copy(
                src_ref=src_f,
                dst_ref=out_ref.at[pl.ds(of * m_per, half)],
                send_sem=send_f.at[h],
                recv_sem=recv_f.at[h],
                device_id=(right,),
                device_id_type=pl.DeviceIdType.MESH,
            )
            rdma_b = pltpu.make_async_remote_copy(
                src_ref=src_b,
                dst_ref=out_ref.at[pl.ds(ob * m_per + half, half)],
                send_sem=send_b.at[h],
                recv_sem=recv_b.at[h],
                device_id=(left,),
                device_id_type=pl.DeviceIdType.MESH,
            )
            rdma_f.start()
            rdma_b.start()
            rdma_f.wait_recv()
            rdma_b.wait_recv()
            rdmas.append((rdma_f, rdma_b))

        for rdma_f, rdma_b in rdmas:
            rdma_f.wait_send()
            rdma_b.wait_send()
        cp.wait()

    return pl.pallas_call(
        body,
        out_shape=jax.ShapeDtypeStruct((N_DEV * m_per, n), x.dtype),
        in_specs=[
            pl.BlockSpec(memory_space=pltpu.MemorySpace.SMEM),
            pl.BlockSpec(memory_space=pltpu.MemorySpace.HBM),
        ],
        out_specs=pl.BlockSpec(memory_space=pltpu.MemorySpace.HBM),
        scratch_shapes=[
            pltpu.SemaphoreType.DMA((N_DEV - 1,)),
            pltpu.SemaphoreType.DMA((N_DEV - 1,)),
            pltpu.SemaphoreType.DMA((N_DEV - 1,)),
            pltpu.SemaphoreType.DMA((N_DEV - 1,)),
            pltpu.SemaphoreType.DMA,
        ],
        compiler_params=pltpu.CompilerParams(collective_id=0),
    )(meta, x)
